# baseline (speedup 1.0000x reference)
import functools

import jax
import jax.numpy as jnp
from jax import lax
from jax.experimental import pallas as pl
from jax.experimental.pallas import tpu as pltpu
from jax.experimental.pallas import tpu_sc as plsc

_POOL_PAD = 128
_NEG = -3e38
_LANES = 16
_NCORES = 2
_NSUB = 16
_CHUNK = 48


def _sc_seqsum_body(x_hbm, out_hbm, acc_ref, buf0, buf1, tail_ref,
                    sem0, sem1, *, seq, d, bpt, boff, ncores):
    nj = d // _LANES
    cpb = seq // _CHUNK
    tail = seq - cpb * _CHUNK
    ntasks = bpt * cpb
    wid = lax.axis_index("s") * ncores + lax.axis_index("c")
    base = wid * bpt
    bufs = (buf0, buf1)
    sems = (sem0, sem1)

    half = nj // 2
    zerosh = tuple(jnp.zeros((_LANES,), jnp.float32) for _ in range(half))

    def zero_body(bi, carry):
        for j in range(nj):
            acc_ref[bi, pl.ds(_LANES * j, _LANES)] = zerosh[0]
        return carry

    lax.fori_loop(0, bpt, zero_body, 0)

    def src(k):
        bi = k // cpb
        r0 = (k % cpb) * _CHUNK
        return x_hbm.at[boff + base + bi, pl.ds(r0, _CHUNK)]

    def issue(k, par):
        pltpu.async_copy(src(k), bufs[par], sems[par])

    def consume(k, par):
        pltpu.make_async_copy(src(k), bufs[par], sems[par]).wait()
        bi = k // cpb
        for h in range(2):
            j0 = h * half

            def row_body(r, a, par=par, j0=j0):
                return tuple(
                    a[t] + bufs[par][r, pl.ds(_LANES * (j0 + t), _LANES)]
                    for t in range(half))

            accs = lax.fori_loop(0, _CHUNK, row_body, zerosh)
            for t in range(half):
                sl = pl.ds(_LANES * (j0 + t), _LANES)
                acc_ref[bi, sl] = acc_ref[bi, sl] + accs[t]

    issue(0, 0)
    issue(1, 1)

    def main_body(kk, carry):
        k = kk * 2
        consume(k, 0)
        issue(k + 2, 0)
        consume(k + 1, 1)
        issue(k + 3, 1)
        return carry

    lax.fori_loop(0, (ntasks - 2) // 2, main_body, 0)
    consume(ntasks - 2, 0)
    consume(ntasks - 1, 1)

    if tail:
        pltpu.async_copy(
            x_hbm.at[pl.ds(boff + base, bpt), pl.ds(cpb * _CHUNK, tail)],
            tail_ref, sems[0]).wait()

        def tail_body(bi, carry):
            for j in range(nj):
                sl = pl.ds(_LANES * j, _LANES)
                a = acc_ref[bi, sl]
                for r in range(tail):
                    a = a + tail_ref[bi, r, sl]
                acc_ref[bi, sl] = a
            return carry

        lax.fori_loop(0, bpt, tail_body, 0)

    pltpu.sync_copy(acc_ref, out_hbm.at[pl.ds(base, bpt)])


def _tc_seqsum_body(x_ref, out_ref):
    out_ref[...] = jnp.sum(x_ref[...], axis=1)


def _sim_topk_body(xlo_ref, xhi0_ref, xhi1_ref, pk_ref, out_ref,
                   *, pool, seq, top_k):
    xsum = jnp.concatenate(
        [xlo_ref[...], xhi0_ref[...], xhi1_ref[...]], axis=0)
    xm = xsum * jnp.float32(1.0 / seq)
    b = xm.shape[0]
    ss = jnp.sum(xm * xm, axis=1, keepdims=True)
    xn = xm * lax.rsqrt(jnp.maximum(ss, 1e-12))
    pk = pk_ref[...]
    ps = jnp.sum(pk * pk, axis=1, keepdims=True)
    pn = pk * lax.rsqrt(jnp.maximum(ps, 1e-12))
    sim = lax.dot_general(xn, pn, (((1,), (1,)), ((), ())),
                          preferred_element_type=jnp.float32)
    work = jnp.concatenate(
        [sim, jnp.full((b, _POOL_PAD - pool), _NEG, jnp.float32)], axis=1)
    col = lax.broadcasted_iota(jnp.int32, (b, _POOL_PAD), 1)
    hist2d = jnp.zeros((b, _POOL_PAD), jnp.int32)
    for _ in range(top_k):
        m = jnp.max(work, axis=1, keepdims=True)
        cand = jnp.where(work == m, col, jnp.int32(1 << 30))
        a = jnp.min(cand, axis=1, keepdims=True)
        pick = col == a
        hist2d = hist2d + pick.astype(jnp.int32)
        work = jnp.where(pick, _NEG, work)
    hist = jnp.sum(hist2d, axis=0, keepdims=True)
    colr = lax.broadcasted_iota(jnp.int32, (1, _POOL_PAD), 1)
    key = hist * 256 + (255 - colr)
    for t in range(top_k):
        m = jnp.max(key)
        out_ref[t] = 255 - (m % 256)
        key = jnp.where(key == m, jnp.int32(-1), key)


def _gather_body(ids_ref, prompt_ref, out_ref, *, top_k, length):
    blk = out_ref.shape[0]
    for t in range(top_k):
        row = prompt_ref[pl.ds(ids_ref[t], 1)]
        out_ref[:, t * length:(t + 1) * length, :] = jnp.broadcast_to(
            row, (blk, length, row.shape[2]))


def kernel(x_embed, prompt, prompt_key):
    b, s, d = x_embed.shape
    pool, length, _ = prompt.shape
    top_k = 8
    bsc = b // 2
    btc = b - bsc
    bpt = bsc // (_NCORES * _NSUB)

    bpc = bsc // 2
    bpt = bpc // _NSUB
    xsum_his = []
    for ci in range(2):
        mesh = plsc.VectorSubcoreMesh(
            core_axis_name="c", subcore_axis_name="s", num_cores=1)
        sc_seqsum = functools.partial(
            pl.kernel,
            out_type=jax.ShapeDtypeStruct((bpc, d), jnp.float32),
            mesh=mesh,
            scratch_types=[
                pltpu.VMEM((bpt, d), jnp.float32),
                pltpu.VMEM((_CHUNK, d), jnp.float32),
                pltpu.VMEM((_CHUNK, d), jnp.float32),
                pltpu.VMEM((bpt, s % _CHUNK, d), jnp.float32),
                pltpu.SemaphoreType.DMA,
                pltpu.SemaphoreType.DMA,
            ],
        )(functools.partial(_sc_seqsum_body, seq=s, d=d, bpt=bpt,
                            boff=btc + ci * bpc, ncores=1))
        xsum_his.append(sc_seqsum(x_embed))

    tblk = 16
    xsum_lo = pl.pallas_call(
        _tc_seqsum_body,
        grid=(btc // tblk,),
        in_specs=[pl.BlockSpec((tblk, s, d), lambda i: (i, 0, 0))],
        out_specs=pl.BlockSpec((tblk, d), lambda i: (i, 0)),
        out_shape=jax.ShapeDtypeStruct((btc, d), jnp.float32),
    )(x_embed)

    ids = pl.pallas_call(
        functools.partial(_sim_topk_body, pool=pool, seq=s, top_k=top_k),
        in_specs=[
            pl.BlockSpec((btc, d), lambda: (0, 0)),
            pl.BlockSpec((bpc, d), lambda: (0, 0)),
            pl.BlockSpec((bpc, d), lambda: (0, 0)),
            pl.BlockSpec((pool, d), lambda: (0, 0)),
        ],
        out_specs=pl.BlockSpec(memory_space=pltpu.SMEM),
        out_shape=jax.ShapeDtypeStruct((top_k,), jnp.int32),
    )(xsum_lo, xsum_his[0], xsum_his[1], prompt_key)

    gblk = 32
    out = pl.pallas_call(
        functools.partial(_gather_body, top_k=top_k, length=length),
        grid=(b // gblk,),
        in_specs=[
            pl.BlockSpec(memory_space=pltpu.SMEM),
            pl.BlockSpec((pool, length, d), lambda i: (0, 0, 0)),
        ],
        out_specs=pl.BlockSpec((gblk, top_k * length, d), lambda i: (i, 0, 0)),
        out_shape=jax.ShapeDtypeStruct((b, top_k * length, d), jnp.float32),
    )(ids, prompt)
    return out


# speedup vs baseline: 1.1633x; 1.1633x over previous
import functools

import jax
import jax.numpy as jnp
from jax import lax
from jax.experimental import pallas as pl
from jax.experimental.pallas import tpu as pltpu
from jax.experimental.pallas import tpu_sc as plsc

_POOL_PAD = 128
_NEG = -3e38
_LANES = 16
_NCORES = 2
_NSUB = 16
_CHUNK = 48


def _sc_seqsum_body(x_hbm, out_hbm, acc_ref, buf0, buf1, tail_ref,
                    sem0, sem1, *, seq, d, bpt, boff, ncores):
    nj = d // _LANES
    cpb = seq // _CHUNK
    tail = seq - cpb * _CHUNK
    ntasks = bpt * cpb
    wid = lax.axis_index("s") * ncores + lax.axis_index("c")
    base = wid * bpt
    bufs = (buf0, buf1)
    sems = (sem0, sem1)

    half = nj // 2
    zerosh = tuple(jnp.zeros((_LANES,), jnp.float32) for _ in range(half))

    def zero_body(bi, carry):
        for j in range(nj):
            acc_ref[bi, pl.ds(_LANES * j, _LANES)] = zerosh[0]
        return carry

    lax.fori_loop(0, bpt, zero_body, 0)

    def src(k):
        bi = k // cpb
        r0 = (k % cpb) * _CHUNK
        return x_hbm.at[boff + base + bi, pl.ds(r0, _CHUNK)]

    def issue(k, par):
        pltpu.async_copy(src(k), bufs[par], sems[par])

    def consume(k, par):
        pltpu.make_async_copy(src(k), bufs[par], sems[par]).wait()
        bi = k // cpb
        for h in range(2):
            j0 = h * half

            def row_body(r, a, par=par, j0=j0):
                return tuple(
                    a[t] + bufs[par][r, pl.ds(_LANES * (j0 + t), _LANES)]
                    for t in range(half))

            accs = lax.fori_loop(0, _CHUNK, row_body, zerosh)
            for t in range(half):
                sl = pl.ds(_LANES * (j0 + t), _LANES)
                acc_ref[bi, sl] = acc_ref[bi, sl] + accs[t]

    issue(0, 0)
    issue(1, 1)

    def main_body(kk, carry):
        k = kk * 2
        consume(k, 0)
        issue(k + 2, 0)
        consume(k + 1, 1)
        issue(k + 3, 1)
        return carry

    lax.fori_loop(0, (ntasks - 2) // 2, main_body, 0)
    consume(ntasks - 2, 0)
    consume(ntasks - 1, 1)

    if tail:
        pltpu.async_copy(
            x_hbm.at[pl.ds(boff + base, bpt), pl.ds(cpb * _CHUNK, tail)],
            tail_ref, sems[0]).wait()

        def tail_body(bi, carry):
            for j in range(nj):
                sl = pl.ds(_LANES * j, _LANES)
                a = acc_ref[bi, sl]
                for r in range(tail):
                    a = a + tail_ref[bi, r, sl]
                acc_ref[bi, sl] = a
            return carry

        lax.fori_loop(0, bpt, tail_body, 0)

    pltpu.sync_copy(acc_ref, out_hbm.at[pl.ds(base, bpt)])


def _tc_seqsum_body(x_ref, out_ref):
    out_ref[...] = jnp.sum(x_ref[...], axis=1)


def _sim_topk_body(xlo_ref, xhi_ref, pk_ref, out_ref, *, pool, seq, top_k):
    xsum = jnp.concatenate([xlo_ref[...], xhi_ref[...]], axis=0)
    xm = xsum * jnp.float32(1.0 / seq)
    b = xm.shape[0]
    ss = jnp.sum(xm * xm, axis=1, keepdims=True)
    xn = xm * lax.rsqrt(jnp.maximum(ss, 1e-12))
    pk = pk_ref[...]
    ps = jnp.sum(pk * pk, axis=1, keepdims=True)
    pn = pk * lax.rsqrt(jnp.maximum(ps, 1e-12))
    sim = lax.dot_general(xn, pn, (((1,), (1,)), ((), ())),
                          preferred_element_type=jnp.float32)
    work = jnp.concatenate(
        [sim, jnp.full((b, _POOL_PAD - pool), _NEG, jnp.float32)], axis=1)
    col = lax.broadcasted_iota(jnp.int32, (b, _POOL_PAD), 1)
    hist2d = jnp.zeros((b, _POOL_PAD), jnp.int32)
    for _ in range(top_k):
        m = jnp.max(work, axis=1, keepdims=True)
        cand = jnp.where(work == m, col, jnp.int32(1 << 30))
        a = jnp.min(cand, axis=1, keepdims=True)
        pick = col == a
        hist2d = hist2d + pick.astype(jnp.int32)
        work = jnp.where(pick, _NEG, work)
    hist = jnp.sum(hist2d, axis=0, keepdims=True)
    colr = lax.broadcasted_iota(jnp.int32, (1, _POOL_PAD), 1)
    key = hist * 256 + (255 - colr)
    for t in range(top_k):
        m = jnp.max(key)
        out_ref[t] = 255 - (m % 256)
        key = jnp.where(key == m, jnp.int32(-1), key)


def _gather_body(ids_ref, prompt_ref, out_ref, *, top_k, length):
    blk = out_ref.shape[0]
    for t in range(top_k):
        row = prompt_ref[pl.ds(ids_ref[t], 1)]
        out_ref[:, t * length:(t + 1) * length, :] = jnp.broadcast_to(
            row, (blk, length, row.shape[2]))


def kernel(x_embed, prompt, prompt_key):
    b, s, d = x_embed.shape
    pool, length, _ = prompt.shape
    top_k = 8
    bsc = b // 4
    btc = b - bsc
    bpt = bsc // (_NCORES * _NSUB)

    bpt = bsc // (_NCORES * _NSUB)
    mesh = plsc.VectorSubcoreMesh(core_axis_name="c", subcore_axis_name="s")
    sc_seqsum = functools.partial(
        pl.kernel,
        out_type=jax.ShapeDtypeStruct((bsc, d), jnp.float32),
        mesh=mesh,
        scratch_types=[
            pltpu.VMEM((bpt, d), jnp.float32),
            pltpu.VMEM((_CHUNK, d), jnp.float32),
            pltpu.VMEM((_CHUNK, d), jnp.float32),
            pltpu.VMEM((bpt, s % _CHUNK, d), jnp.float32),
            pltpu.SemaphoreType.DMA,
            pltpu.SemaphoreType.DMA,
        ],
    )(functools.partial(_sc_seqsum_body, seq=s, d=d, bpt=bpt,
                        boff=btc, ncores=_NCORES))
    xsum_hi = sc_seqsum(x_embed)

    tblk = 16
    xsum_lo = pl.pallas_call(
        _tc_seqsum_body,
        grid=(btc // tblk,),
        in_specs=[pl.BlockSpec((tblk, s, d), lambda i: (i, 0, 0))],
        out_specs=pl.BlockSpec((tblk, d), lambda i: (i, 0)),
        out_shape=jax.ShapeDtypeStruct((btc, d), jnp.float32),
    )(x_embed)

    ids = pl.pallas_call(
        functools.partial(_sim_topk_body, pool=pool, seq=s, top_k=top_k),
        in_specs=[
            pl.BlockSpec((btc, d), lambda: (0, 0)),
            pl.BlockSpec((bsc, d), lambda: (0, 0)),
            pl.BlockSpec((pool, d), lambda: (0, 0)),
        ],
        out_specs=pl.BlockSpec(memory_space=pltpu.SMEM),
        out_shape=jax.ShapeDtypeStruct((top_k,), jnp.int32),
    )(xsum_lo, xsum_hi, prompt_key)

    gblk = 32
    out = pl.pallas_call(
        functools.partial(_gather_body, top_k=top_k, length=length),
        grid=(b // gblk,),
        in_specs=[
            pl.BlockSpec(memory_space=pltpu.SMEM),
            pl.BlockSpec((pool, length, d), lambda i: (0, 0, 0)),
        ],
        out_specs=pl.BlockSpec((gblk, top_k * length, d), lambda i: (i, 0, 0)),
        out_shape=jax.ShapeDtypeStruct((b, top_k * length, d), jnp.float32),
    )(ids, prompt)
    return out


# speedup vs baseline: 1.1689x; 1.0048x over previous
import functools

import jax
import jax.numpy as jnp
from jax import lax
from jax.experimental import pallas as pl
from jax.experimental.pallas import tpu as pltpu
from jax.experimental.pallas import tpu_sc as plsc

_POOL_PAD = 128
_NEG = -3e38
_LANES = 16
_NCORES = 2
_NSUB = 16
_CHUNK = 64


def _sc_seqsum_body(x_hbm, out_hbm, acc_ref, buf0, buf1, tail_ref,
                    sem0, sem1, *, seq, d, bpt, boff, ncores):
    nj = d // _LANES
    cpb = seq // _CHUNK
    tail = seq - cpb * _CHUNK
    ntasks = bpt * cpb
    wid = lax.axis_index("s") * ncores + lax.axis_index("c")
    base = wid * bpt
    bufs = (buf0, buf1)
    sems = (sem0, sem1)

    half = nj // 2
    zerosh = tuple(jnp.zeros((_LANES,), jnp.float32) for _ in range(half))

    def zero_body(bi, carry):
        for j in range(nj):
            acc_ref[bi, pl.ds(_LANES * j, _LANES)] = zerosh[0]
        return carry

    lax.fori_loop(0, bpt, zero_body, 0)

    def src(k):
        bi = k // cpb
        r0 = (k % cpb) * _CHUNK
        return x_hbm.at[boff + base + bi, pl.ds(r0, _CHUNK)]

    def issue(k, par):
        pltpu.async_copy(src(k), bufs[par], sems[par])

    def consume(k, par):
        pltpu.make_async_copy(src(k), bufs[par], sems[par]).wait()
        bi = k // cpb
        for h in range(2):
            j0 = h * half

            def row_body(r, a, par=par, j0=j0):
                return tuple(
                    a[t] + bufs[par][r, pl.ds(_LANES * (j0 + t), _LANES)]
                    for t in range(half))

            accs = lax.fori_loop(0, _CHUNK, row_body, zerosh)
            for t in range(half):
                sl = pl.ds(_LANES * (j0 + t), _LANES)
                acc_ref[bi, sl] = acc_ref[bi, sl] + accs[t]

    issue(0, 0)
    issue(1, 1)

    def main_body(kk, carry):
        k = kk * 2
        consume(k, 0)
        issue(k + 2, 0)
        consume(k + 1, 1)
        issue(k + 3, 1)
        return carry

    lax.fori_loop(0, (ntasks - 2) // 2, main_body, 0)
    consume(ntasks - 2, 0)
    consume(ntasks - 1, 1)

    if tail:
        pltpu.async_copy(
            x_hbm.at[pl.ds(boff + base, bpt), pl.ds(cpb * _CHUNK, tail)],
            tail_ref, sems[0]).wait()

        def tail_body(bi, carry):
            for j in range(nj):
                sl = pl.ds(_LANES * j, _LANES)
                a = acc_ref[bi, sl]
                for r in range(tail):
                    a = a + tail_ref[bi, r, sl]
                acc_ref[bi, sl] = a
            return carry

        lax.fori_loop(0, bpt, tail_body, 0)

    pltpu.sync_copy(acc_ref, out_hbm.at[pl.ds(base, bpt)])


def _tc_seqsum_body(x_ref, out_ref):
    out_ref[...] = jnp.sum(x_ref[...], axis=1)


def _sim_topk_body(xlo_ref, xhi_ref, pk_ref, out_ref, *, pool, seq, top_k):
    xsum = jnp.concatenate([xlo_ref[...], xhi_ref[...]], axis=0)
    xm = xsum * jnp.float32(1.0 / seq)
    b = xm.shape[0]
    ss = jnp.sum(xm * xm, axis=1, keepdims=True)
    xn = xm * lax.rsqrt(jnp.maximum(ss, 1e-12))
    pk = pk_ref[...]
    ps = jnp.sum(pk * pk, axis=1, keepdims=True)
    pn = pk * lax.rsqrt(jnp.maximum(ps, 1e-12))
    sim = lax.dot_general(xn, pn, (((1,), (1,)), ((), ())),
                          preferred_element_type=jnp.float32)
    work = jnp.concatenate(
        [sim, jnp.full((b, _POOL_PAD - pool), _NEG, jnp.float32)], axis=1)
    col = lax.broadcasted_iota(jnp.int32, (b, _POOL_PAD), 1)
    hist2d = jnp.zeros((b, _POOL_PAD), jnp.int32)
    for _ in range(top_k):
        m = jnp.max(work, axis=1, keepdims=True)
        cand = jnp.where(work == m, col, jnp.int32(1 << 30))
        a = jnp.min(cand, axis=1, keepdims=True)
        pick = col == a
        hist2d = hist2d + pick.astype(jnp.int32)
        work = jnp.where(pick, _NEG, work)
    hist = jnp.sum(hist2d, axis=0, keepdims=True)
    colr = lax.broadcasted_iota(jnp.int32, (1, _POOL_PAD), 1)
    key = hist * 256 + (255 - colr)
    for t in range(top_k):
        m = jnp.max(key)
        out_ref[t] = 255 - (m % 256)
        key = jnp.where(key == m, jnp.int32(-1), key)


def _gather_body(ids_ref, prompt_ref, out_ref, *, top_k, length):
    blk = out_ref.shape[0]
    for t in range(top_k):
        row = prompt_ref[pl.ds(ids_ref[t], 1)]
        out_ref[:, t * length:(t + 1) * length, :] = jnp.broadcast_to(
            row, (blk, length, row.shape[2]))


def kernel(x_embed, prompt, prompt_key):
    b, s, d = x_embed.shape
    pool, length, _ = prompt.shape
    top_k = 8
    bsc = b // 4
    btc = b - bsc
    bpt = bsc // (_NCORES * _NSUB)

    bpt = bsc // (_NCORES * _NSUB)
    mesh = plsc.VectorSubcoreMesh(core_axis_name="c", subcore_axis_name="s")
    sc_seqsum = functools.partial(
        pl.kernel,
        out_type=jax.ShapeDtypeStruct((bsc, d), jnp.float32),
        mesh=mesh,
        scratch_types=[
            pltpu.VMEM((bpt, d), jnp.float32),
            pltpu.VMEM((_CHUNK, d), jnp.float32),
            pltpu.VMEM((_CHUNK, d), jnp.float32),
            pltpu.VMEM((bpt, s % _CHUNK, d), jnp.float32),
            pltpu.SemaphoreType.DMA,
            pltpu.SemaphoreType.DMA,
        ],
    )(functools.partial(_sc_seqsum_body, seq=s, d=d, bpt=bpt,
                        boff=btc, ncores=_NCORES))
    xsum_hi = sc_seqsum(x_embed)

    tblk = 32
    xsum_lo = pl.pallas_call(
        _tc_seqsum_body,
        grid=(btc // tblk,),
        in_specs=[pl.BlockSpec((tblk, s, d), lambda i: (i, 0, 0))],
        out_specs=pl.BlockSpec((tblk, d), lambda i: (i, 0)),
        out_shape=jax.ShapeDtypeStruct((btc, d), jnp.float32),
    )(x_embed)

    ids = pl.pallas_call(
        functools.partial(_sim_topk_body, pool=pool, seq=s, top_k=top_k),
        in_specs=[
            pl.BlockSpec((btc, d), lambda: (0, 0)),
            pl.BlockSpec((bsc, d), lambda: (0, 0)),
            pl.BlockSpec((pool, d), lambda: (0, 0)),
        ],
        out_specs=pl.BlockSpec(memory_space=pltpu.SMEM),
        out_shape=jax.ShapeDtypeStruct((top_k,), jnp.int32),
    )(xsum_lo, xsum_hi, prompt_key)

    gblk = 32
    out = pl.pallas_call(
        functools.partial(_gather_body, top_k=top_k, length=length),
        grid=(b // gblk,),
        in_specs=[
            pl.BlockSpec(memory_space=pltpu.SMEM),
            pl.BlockSpec((pool, length, d), lambda i: (0, 0, 0)),
        ],
        out_specs=pl.BlockSpec((gblk, top_k * length, d), lambda i: (i, 0, 0)),
        out_shape=jax.ShapeDtypeStruct((b, top_k * length, d), jnp.float32),
    )(ids, prompt)
    return out
